# baseline (speedup 1.0000x reference)
import jax
import jax.numpy as jnp
from jax.experimental import pallas as pl
from jax.experimental.pallas import tpu as pltpu

B, C, H, W = 32, 96, 224, 224
R, E, HID, K = 4, 8, 64, 2
HW = H * W
NCHUNK = 8
CHUNK = HW // NCHUNK


def _pool_body(x_ref, out_ref):
    j = pl.program_id(1)
    s = jnp.sum(x_ref[0], axis=1)

    @pl.when(j == 0)
    def _init():
        out_ref[0, 0, :] = s

    @pl.when(j != 0)
    def _acc():
        out_ref[0, 0, :] = out_ref[0, 0, :] + s


def _gate_body(p_ref, w1t_ref, b1_ref, w2bd_ref, b2_ref, idx_ref, sc_ref,
               logit_ref):
    pooled = p_ref[:, 0, :] * (1.0 / HW)
    h = jnp.maximum(
        jnp.dot(pooled, w1t_ref[...], preferred_element_type=jnp.float32)
        + b1_ref[...], 0.0)
    logits = (jnp.dot(h, w2bd_ref[...], preferred_element_type=jnp.float32)
              + b2_ref[...])
    logit_ref[...] = logits

    iota = jax.lax.broadcasted_iota(jnp.int32, (B, E), 1)
    idx_cols = []
    sc_cols = []
    for r in range(R):
        lr = logits[:, r * E:(r + 1) * E]
        v1 = jnp.max(lr, axis=1, keepdims=True)
        i1 = jnp.min(jnp.where(lr == v1, iota, E), axis=1, keepdims=True)
        masked = jnp.where(iota == i1, -jnp.inf, lr)
        v2 = jnp.max(masked, axis=1, keepdims=True)
        i2 = jnp.min(jnp.where(masked == v2, iota, E), axis=1, keepdims=True)
        t = jnp.exp(v2 - v1)
        s1 = 1.0 / (1.0 + t)
        idx_cols += [i1, i2]
        sc_cols += [s1, 1.0 - s1]
    idx_ref[...] = jnp.concatenate(idx_cols, axis=1)
    sc_ref[...] = jnp.concatenate(sc_cols, axis=1)


def kernel(x, W1, b1, W2, b2):
    xf = x.reshape(B, C, HW)
    pooled_sum = pl.pallas_call(
        _pool_body,
        grid=(B, NCHUNK),
        in_specs=[pl.BlockSpec((1, C, CHUNK), lambda b, j: (b, 0, j))],
        out_specs=pl.BlockSpec((1, 1, C), lambda b, j: (b, 0, 0)),
        out_shape=jax.ShapeDtypeStruct((B, 1, C), jnp.float32),
        compiler_params=pltpu.CompilerParams(
            dimension_semantics=("parallel", "arbitrary")),
    )(xf)

    w1t = W1.reshape(R * HID, C).T
    b1f = b1.reshape(1, R * HID)
    w2bd = jnp.zeros((R * HID, R * E), jnp.float32)
    for r in range(R):
        w2bd = w2bd.at[r * HID:(r + 1) * HID, r * E:(r + 1) * E].set(W2[r].T)
    b2f = b2.reshape(1, R * E)

    idx2d, sc2d, logits2d = pl.pallas_call(
        _gate_body,
        out_shape=(
            jax.ShapeDtypeStruct((B, R * K), jnp.int32),
            jax.ShapeDtypeStruct((B, R * K), jnp.float32),
            jax.ShapeDtypeStruct((B, R * E), jnp.float32),
        ),
    )(pooled_sum, w1t, b1f, w2bd, b2f)

    return (idx2d.reshape(B, R, K), sc2d.reshape(B, R, K),
            logits2d.reshape(B, R, E))


# speedup vs baseline: 1.1016x; 1.1016x over previous
import jax
import jax.numpy as jnp
from jax.experimental import pallas as pl
from jax.experimental.pallas import tpu as pltpu

B, C, H, W = 32, 96, 224, 224
R, E, HID, K = 4, 8, 64, 2
HW = H * W
NCHUNK = 8
CHUNK = HW // NCHUNK


XROWS = 48


def _pool_body(x_ref, out_ref):
    out_ref[...] = jnp.sum(x_ref[...], axis=1, keepdims=True)


def _gate_body(p_ref, w1t_ref, b1_ref, w2bd_ref, b2_ref, idx_ref, sc_ref,
               logit_ref):
    pooled = p_ref[:, 0, :] * (1.0 / HW)
    h = jnp.maximum(
        jnp.dot(pooled, w1t_ref[...], preferred_element_type=jnp.float32)
        + b1_ref[...], 0.0)
    logits = (jnp.dot(h, w2bd_ref[...], preferred_element_type=jnp.float32)
              + b2_ref[...])
    logit_ref[...] = logits

    iota = jax.lax.broadcasted_iota(jnp.int32, (B, E), 1)
    idx_cols = []
    sc_cols = []
    for r in range(R):
        lr = logits[:, r * E:(r + 1) * E]
        v1 = jnp.max(lr, axis=1, keepdims=True)
        i1 = jnp.min(jnp.where(lr == v1, iota, E), axis=1, keepdims=True)
        masked = jnp.where(iota == i1, -jnp.inf, lr)
        v2 = jnp.max(masked, axis=1, keepdims=True)
        i2 = jnp.min(jnp.where(masked == v2, iota, E), axis=1, keepdims=True)
        t = jnp.exp(v2 - v1)
        s1 = 1.0 / (1.0 + t)
        idx_cols += [i1, i2]
        sc_cols += [s1, 1.0 - s1]
    idx_ref[...] = jnp.concatenate(idx_cols, axis=1)
    sc_ref[...] = jnp.concatenate(sc_cols, axis=1)


def kernel(x, W1, b1, W2, b2):
    xf = x.reshape(B * C, HW)
    pooled_sum = pl.pallas_call(
        _pool_body,
        grid=(B * C // XROWS,),
        in_specs=[pl.BlockSpec((XROWS, HW), lambda i: (i, 0))],
        out_specs=pl.BlockSpec((XROWS, 1), lambda i: (i, 0)),
        out_shape=jax.ShapeDtypeStruct((B * C, 1), jnp.float32),
        compiler_params=pltpu.CompilerParams(
            dimension_semantics=("parallel",)),
    )(xf)
    pooled_sum = pooled_sum.reshape(B, 1, C)

    w1t = W1.reshape(R * HID, C).T
    b1f = b1.reshape(1, R * HID)
    w2bd = jnp.zeros((R * HID, R * E), jnp.float32)
    for r in range(R):
        w2bd = w2bd.at[r * HID:(r + 1) * HID, r * E:(r + 1) * E].set(W2[r].T)
    b2f = b2.reshape(1, R * E)

    idx2d, sc2d, logits2d = pl.pallas_call(
        _gate_body,
        out_shape=(
            jax.ShapeDtypeStruct((B, R * K), jnp.int32),
            jax.ShapeDtypeStruct((B, R * K), jnp.float32),
            jax.ShapeDtypeStruct((B, R * E), jnp.float32),
        ),
    )(pooled_sum, w1t, b1f, w2bd, b2f)

    return (idx2d.reshape(B, R, K), sc2d.reshape(B, R, K),
            logits2d.reshape(B, R, E))


# speedup vs baseline: 4.3024x; 3.9057x over previous
import jax
import jax.numpy as jnp
from jax.experimental import pallas as pl
from jax.experimental.pallas import tpu as pltpu

B, C, H, W = 32, 96, 224, 224
R, E, HID, K = 4, 8, 64, 2
HW = H * W
NCHUNK = 8
CHUNK = HW // NCHUNK


def _pool_body(x_ref, out_ref):
    out_ref[0, 0, :] = jnp.sum(x_ref[0], axis=(1, 2))


def _gate_body(p_ref, w1t_ref, b1_ref, w2bd_ref, b2_ref, idx_ref, sc_ref,
               logit_ref):
    pooled = p_ref[:, 0, :] * (1.0 / HW)
    h = jnp.maximum(
        jnp.dot(pooled, w1t_ref[...], preferred_element_type=jnp.float32)
        + b1_ref[...], 0.0)
    logits = (jnp.dot(h, w2bd_ref[...], preferred_element_type=jnp.float32)
              + b2_ref[...])
    logit_ref[...] = logits

    iota = jax.lax.broadcasted_iota(jnp.int32, (B, E), 1)
    idx_cols = []
    sc_cols = []
    for r in range(R):
        lr = logits[:, r * E:(r + 1) * E]
        v1 = jnp.max(lr, axis=1, keepdims=True)
        i1 = jnp.min(jnp.where(lr == v1, iota, E), axis=1, keepdims=True)
        masked = jnp.where(iota == i1, -jnp.inf, lr)
        v2 = jnp.max(masked, axis=1, keepdims=True)
        i2 = jnp.min(jnp.where(masked == v2, iota, E), axis=1, keepdims=True)
        t = jnp.exp(v2 - v1)
        s1 = 1.0 / (1.0 + t)
        idx_cols += [i1, i2]
        sc_cols += [s1, 1.0 - s1]
    idx_ref[...] = jnp.concatenate(idx_cols, axis=1)
    sc_ref[...] = jnp.concatenate(sc_cols, axis=1)


def kernel(x, W1, b1, W2, b2):
    pooled_sum = pl.pallas_call(
        _pool_body,
        grid=(B,),
        in_specs=[pl.BlockSpec((1, C, H, W), lambda b: (b, 0, 0, 0))],
        out_specs=pl.BlockSpec((1, 1, C), lambda b: (b, 0, 0)),
        out_shape=jax.ShapeDtypeStruct((B, 1, C), jnp.float32),
        compiler_params=pltpu.CompilerParams(
            dimension_semantics=("parallel",)),
    )(x)

    w1t = W1.reshape(R * HID, C).T
    b1f = b1.reshape(1, R * HID)
    w2bd = jnp.zeros((R * HID, R * E), jnp.float32)
    for r in range(R):
        w2bd = w2bd.at[r * HID:(r + 1) * HID, r * E:(r + 1) * E].set(W2[r].T)
    b2f = b2.reshape(1, R * E)

    idx2d, sc2d, logits2d = pl.pallas_call(
        _gate_body,
        out_shape=(
            jax.ShapeDtypeStruct((B, R * K), jnp.int32),
            jax.ShapeDtypeStruct((B, R * K), jnp.float32),
            jax.ShapeDtypeStruct((B, R * E), jnp.float32),
        ),
    )(pooled_sum, w1t, b1f, w2bd, b2f)

    return (idx2d.reshape(B, R, K), sc2d.reshape(B, R, K),
            logits2d.reshape(B, R, E))
